# Initial kernel scaffold; baseline (speedup 1.0000x reference)
import jax
import jax.numpy as jnp
from jax.experimental import pallas as pl


def kernel(x, edge_index, W1, b1, g1, W2, b2, g2):
    raise NotImplementedError("write your pallas kernel here")


# speedup vs baseline: 22.3215x; 22.3215x over previous
import functools

import jax
import jax.numpy as jnp
from jax import lax
from jax.experimental import pallas as pl
from jax.experimental.pallas import tpu as pltpu
from jax.experimental.pallas import tpu_sc as plsc

N = 10000
NP = 10240
D = 128
E = 320000
NC = 2
NS = 16
NW = NC * NS
B = 128
CH = (E + NW * B - 1) // (NW * B)
EP = NW * CH * B
RPT = NP // NS
_BETA = 0.5
_CMIX = 1.0

_mesh = plsc.VectorSubcoreMesh(core_axis_name="c", subcore_axis_name="s")


def _zero16(zb, sid, acc):
    z = jnp.zeros((16,), jnp.float32)
    for r in range(16):
        for c in range(D // 16):
            zb[r, pl.ds(c * 16, 16)] = z

    def body(i, _):
        pltpu.sync_copy(zb, acc.at[pl.ds(sid * RPT + i * 16, 16)])
        return 0

    lax.fori_loop(0, RPT // 16, body, 0)


def _deg_body(dstb, deg1d, dstv, onesb, zb, acc):
    cid = lax.axis_index("c")
    sid = lax.axis_index("s")
    wid = cid * NS + sid
    one = jnp.ones((16,), jnp.float32)
    for r in range(B // 16):
        onesb[pl.ds(r * 16, 16)] = one
    z = jnp.zeros((16,), jnp.float32)
    for r in range(RPT // 16):
        zb[pl.ds(r * 16, 16)] = z
    pltpu.sync_copy(zb, acc.at[pl.ds(sid * RPT, RPT)])
    pltpu.sync_copy(dstb.at[wid], dstv)
    plsc.subcore_barrier()

    def body(j, _):
        pltpu.sync_copy(onesb, acc.at[dstv.at[j]], add=True)
        return 0

    lax.fori_loop(0, CH, body, 0)
    plsc.subcore_barrier()
    pltpu.sync_copy(acc.at[pl.ds(sid * RPT, RPT)],
                    deg1d.at[pl.ds(cid * NP + sid * RPT, RPT)])


_deg_call = functools.partial(
    pl.kernel,
    _deg_body,
    out_type=jax.ShapeDtypeStruct((NC * NP,), jnp.float32),
    mesh=_mesh,
    scratch_types=[
        pltpu.VMEM((CH, B), jnp.int32),
        pltpu.VMEM((B,), jnp.float32),
        pltpu.VMEM((RPT,), jnp.float32),
        pltpu.VMEM_SHARED((NP,), jnp.float32),
    ],
)()


def _scat_body(yp, srcb, dstb, outp, srcv, dstv, rows, zb, acc, gsem):
    cid = lax.axis_index("c")
    sid = lax.axis_index("s")
    wid = cid * NS + sid
    _zero16(zb, sid, acc)
    pltpu.sync_copy(srcb.at[wid], srcv)
    pltpu.sync_copy(dstb.at[wid], dstv)
    plsc.subcore_barrier()

    def body(j, _):
        pltpu.async_copy(yp.at[srcv.at[j]], rows, gsem).wait()
        pltpu.sync_copy(rows, acc.at[dstv.at[j]], add=True)
        return 0

    lax.fori_loop(0, CH, body, 0)
    plsc.subcore_barrier()
    pltpu.sync_copy(acc.at[pl.ds(sid * RPT, RPT)],
                    outp.at[cid, pl.ds(sid * RPT, RPT)])


_scat_call = functools.partial(
    pl.kernel,
    _scat_body,
    out_type=jax.ShapeDtypeStruct((NC, NP, D), jnp.float32),
    mesh=_mesh,
    scratch_types=[
        pltpu.VMEM((CH, B), jnp.int32),
        pltpu.VMEM((CH, B), jnp.int32),
        pltpu.VMEM((B, D), jnp.float32),
        pltpu.VMEM((16, D), jnp.float32),
        pltpu.VMEM_SHARED((NP, D), jnp.float32),
        pltpu.SemaphoreType.DMA,
    ],
)()


def _t1_body(x_ref, w_ref, d0_ref, d1_ref, y_ref, dinv_ref):
    deg = d0_ref[...] + d1_ref[...] + 1.0
    dinv = lax.rsqrt(deg)
    dinv_ref[...] = dinv
    xw = jnp.dot(x_ref[...], w_ref[...],
                 preferred_element_type=jnp.float32,
                 precision=lax.Precision.HIGHEST)
    y_ref[...] = xw * dinv


def _mix(z, g_ref):
    ga = g_ref[0]
    gb = g_ref[1]
    gc = g_ref[2]
    gd = g_ref[3]
    gr = jnp.where(z < 0, ga * z, z)
    gr = jnp.where((z >= 0) & (z < gc), gb * z, gr)
    gr = jnp.where(z >= gc, gd * z, gr)
    return _BETA * z + (_CMIX - _BETA) * gr


def _t2_body(p_ref, y1_ref, dinv_ref, b_ref, g_ref, w_ref, y2_ref):
    z = dinv_ref[...] * (p_ref[0] + p_ref[1] + y1_ref[...]) + b_ref[...][None, :]
    h = _mix(z, g_ref)
    row = lax.broadcasted_iota(jnp.int32, (NP, 1), 0)
    h = jnp.where(row < N, h, 0.0)
    xw2 = jnp.dot(h, w_ref[...],
                  preferred_element_type=jnp.float32,
                  precision=lax.Precision.HIGHEST)
    y2_ref[...] = xw2 * dinv_ref[...]


def _t3_body(p_ref, y2_ref, dinv_ref, b_ref, g_ref, o_ref):
    z = dinv_ref[...] * (p_ref[0] + p_ref[1] + y2_ref[...]) + b_ref[...][None, :]
    o_ref[...] = _mix(z, g_ref)


def kernel(x, edge_index, W1, b1, g1, W2, b2, g2):
    src = edge_index[0]
    dst = edge_index[1]
    padidx = N + (jnp.arange(EP - E, dtype=jnp.int32) % (NP - N))
    srcb = jnp.concatenate([src, padidx]).reshape(NW, CH, B)
    dstb = jnp.concatenate([dst, padidx]).reshape(NW, CH, B)
    xp = jnp.pad(x, ((0, NP - N), (0, 0)))

    degflat = _deg_call(dstb)
    d0 = degflat[:NP].reshape(NP, 1)
    d1 = degflat[NP:].reshape(NP, 1)

    y1, dinv = pl.pallas_call(
        _t1_body,
        out_shape=(jax.ShapeDtypeStruct((NP, D), jnp.float32),
                   jax.ShapeDtypeStruct((NP, 1), jnp.float32)),
    )(xp, W1, d0, d1)

    p1 = _scat_call(y1, srcb, dstb)

    y2 = pl.pallas_call(
        _t2_body,
        out_shape=jax.ShapeDtypeStruct((NP, D), jnp.float32),
    )(p1, y1, dinv, b1, g1, W2)

    p2 = _scat_call(y2, srcb, dstb)

    out = pl.pallas_call(
        _t3_body,
        out_shape=jax.ShapeDtypeStruct((NP, D), jnp.float32),
    )(p2, y2, dinv, b2, g2)

    return out[:N]


# speedup vs baseline: 27.3459x; 1.2251x over previous
import functools

import jax
import jax.numpy as jnp
from jax import lax
from jax.experimental import pallas as pl
from jax.experimental.pallas import tpu as pltpu
from jax.experimental.pallas import tpu_sc as plsc

N = 10000
NP = 10240
D = 128
E = 320000
NC = 2
NS = 16
NW = NC * NS
B = 128
CH = 80
PASSES = 2
CHP = CH // PASSES
EP = NW * CH * B
RPT = NP // NS
_BETA = 0.5
_CMIX = 1.0

@functools.cache
def _mesh():
    return plsc.VectorSubcoreMesh(core_axis_name="c", subcore_axis_name="s",
                                  num_cores=NC, num_subcores=NS)


def _deg_body(dstb, deg1d, dstv, onesb, zb, acc):
    cid = lax.axis_index("c")
    sid = lax.axis_index("s")
    wid = cid * NS + sid
    one = jnp.ones((16,), jnp.float32)
    for r in range(B // 16):
        onesb[pl.ds(r * 16, 16)] = one
    z = jnp.zeros((16,), jnp.float32)
    for r in range(RPT // 16):
        zb[pl.ds(r * 16, 16)] = z
    pltpu.sync_copy(zb, acc.at[pl.ds(sid * RPT, RPT)])
    pltpu.sync_copy(dstb.at[wid], dstv)
    plsc.subcore_barrier()

    def body(j, _):
        pltpu.sync_copy(onesb, acc.at[dstv.at[j]], add=True)
        return 0

    lax.fori_loop(0, CH, body, 0)
    plsc.subcore_barrier()
    pltpu.sync_copy(acc.at[pl.ds(sid * RPT, RPT)],
                    deg1d.at[pl.ds(cid * NP + sid * RPT, RPT)])


def _deg_call(dstb):
    return pl.kernel(
        _deg_body,
        out_type=jax.ShapeDtypeStruct((NC * NP,), jnp.float32),
        mesh=_mesh(),
        scratch_types=[
            pltpu.VMEM((CH, B), jnp.int32),
            pltpu.VMEM((B,), jnp.float32),
            pltpu.VMEM((RPT,), jnp.float32),
            pltpu.VMEM_SHARED((NP,), jnp.float32),
        ],
    )(dstb)


def _scat_body(yp, zhbm, srcb, dstb, outp, srcv, dstv, rows0, rows1, acc, gsem):
    cid = lax.axis_index("c")
    sid = lax.axis_index("s")
    wid = cid * NS + sid
    rs = pl.ds(sid * RPT, RPT)

    @pl.when(cid == 0)
    def _():
        pltpu.sync_copy(yp.at[rs], acc.at[rs])

    @pl.when(cid != 0)
    def _():
        pltpu.sync_copy(zhbm.at[rs], acc.at[rs])

    plsc.subcore_barrier()

    def passloop(p, _):
        pltpu.sync_copy(srcb.at[wid, pl.ds(p * CHP, CHP)], srcv)
        pltpu.sync_copy(dstb.at[wid, pl.ds(p * CHP, CHP)], dstv)
        pltpu.async_copy(yp.at[srcv.at[0]], rows0, gsem)

        def body(i, _):
            j = 2 * i
            pltpu.make_async_copy(yp.at[srcv.at[j]], rows0, gsem).wait()
            pltpu.async_copy(yp.at[srcv.at[j + 1]], rows1, gsem)
            pltpu.sync_copy(rows0, acc.at[dstv.at[j]], add=True)
            pltpu.make_async_copy(yp.at[srcv.at[j + 1]], rows1, gsem).wait()

            @pl.when(j + 2 < CHP)
            def _():
                pltpu.async_copy(yp.at[srcv.at[j + 2]], rows0, gsem)

            pltpu.sync_copy(rows1, acc.at[dstv.at[j + 1]], add=True)
            return 0

        lax.fori_loop(0, CHP // 2, body, 0)
        return 0

    lax.fori_loop(0, PASSES, passloop, 0)
    plsc.subcore_barrier()
    pltpu.sync_copy(acc.at[rs], outp.at[cid, rs])


def _scat_call(yp, zer, srcb, dstb):
    return pl.kernel(
        _scat_body,
        out_type=jax.ShapeDtypeStruct((NC, NP, D), jnp.float32),
        mesh=_mesh(),
        scratch_types=[
            pltpu.VMEM((CHP, B), jnp.int32),
            pltpu.VMEM((CHP, B), jnp.int32),
            pltpu.VMEM((B, D), jnp.float32),
            pltpu.VMEM((B, D), jnp.float32),
            pltpu.VMEM_SHARED((NP, D), jnp.float32),
            pltpu.SemaphoreType.DMA,
        ],
    )(yp, zer, srcb, dstb)


def _t1_body(x_ref, w_ref, d0_ref, d1_ref, y_ref, dinv_ref):
    deg = d0_ref[...] + d1_ref[...] + 1.0
    dinv = lax.rsqrt(deg)
    dinv_ref[...] = dinv
    xw = jnp.dot(x_ref[...], w_ref[...],
                 preferred_element_type=jnp.float32,
                 precision=lax.Precision.HIGHEST)
    y_ref[...] = xw * dinv


def _mix(z, g_ref):
    ga = g_ref[0]
    gb = g_ref[1]
    gc = g_ref[2]
    gd = g_ref[3]
    gr = jnp.where(z < 0, ga * z, z)
    gr = jnp.where((z >= 0) & (z < gc), gb * z, gr)
    gr = jnp.where(z >= gc, gd * z, gr)
    return _BETA * z + (_CMIX - _BETA) * gr


def _t2_body(p_ref, dinv_ref, b_ref, g_ref, w_ref, y2_ref):
    z = dinv_ref[...] * (p_ref[0] + p_ref[1]) + b_ref[...][None, :]
    h = _mix(z, g_ref)
    row = lax.broadcasted_iota(jnp.int32, (NP, 1), 0)
    h = jnp.where(row < N, h, 0.0)
    xw2 = jnp.dot(h, w_ref[...],
                  preferred_element_type=jnp.float32,
                  precision=lax.Precision.HIGHEST)
    y2_ref[...] = xw2 * dinv_ref[...]


def _t3_body(p_ref, dinv_ref, b_ref, g_ref, o_ref):
    z = dinv_ref[...] * (p_ref[0] + p_ref[1]) + b_ref[...][None, :]
    o_ref[...] = _mix(z, g_ref)


def kernel(x, edge_index, W1, b1, g1, W2, b2, g2):
    src = edge_index[0]
    dst = edge_index[1]
    padidx = N + (jnp.arange(EP - E, dtype=jnp.int32) % (NP - N))
    srcb = jnp.concatenate([src, padidx]).reshape(NW, CH, B)
    dstb = jnp.concatenate([dst, padidx]).reshape(NW, CH, B)
    xp = jnp.pad(x, ((0, NP - N), (0, 0)))

    degflat = _deg_call(dstb)
    d0 = degflat[:NP].reshape(NP, 1)
    d1 = degflat[NP:].reshape(NP, 1)

    y1, dinv = pl.pallas_call(
        _t1_body,
        out_shape=(jax.ShapeDtypeStruct((NP, D), jnp.float32),
                   jax.ShapeDtypeStruct((NP, 1), jnp.float32)),
    )(xp, W1, d0, d1)

    zer = jnp.zeros((NP, D), jnp.float32)
    p1 = _scat_call(y1, zer, srcb, dstb)

    y2 = pl.pallas_call(
        _t2_body,
        out_shape=jax.ShapeDtypeStruct((NP, D), jnp.float32),
    )(p1, dinv, b1, g1, W2)

    p2 = _scat_call(y2, zer, srcb, dstb)

    out = pl.pallas_call(
        _t3_body,
        out_shape=jax.ShapeDtypeStruct((NP, D), jnp.float32),
    )(p2, dinv, b2, g2)

    return out[:N]
